# Initial kernel scaffold; baseline (speedup 1.0000x reference)
import jax
import jax.numpy as jnp
from jax.experimental import pallas as pl


def kernel(audio_feats, text_feats, W1, a_src1, a_dst1, W2, a_src2, a_dst2, Wm1, bm1, Wm2, bm2):
    raise NotImplementedError("write your pallas kernel here")


# speedup vs baseline: 125.9065x; 125.9065x over previous
import jax
import jax.numpy as jnp
from jax import lax
from jax.experimental import pallas as pl
from jax.experimental.pallas import tpu as pltpu

_IN = 256
_H = 4
_D = 64
_TA = 200
_TT = 50
_TTP = 64
_NEG = -1e30


def _leaky(x):
    return jnp.where(x >= 0, x, 0.2 * x)


def _gat_fused_kernel(a_ref, t_ref, w1_ref, a1_ref, w2_ref, a2_ref,
                      wm1_ref, bm1_ref, wm2_ref, bm2_ref, out_ref):
    a = a_ref[0]
    t = t_ref[0]
    col_mask = lax.broadcasted_iota(jnp.int32, (_TA, _TTP), 1) < _TT
    row_mask = lax.broadcasted_iota(jnp.int32, (_TTP, 1), 0) < _TT

    def layer(ha_in, ht_in, w_ref, A_ref, activate):
        W = w_ref[...]
        A = A_ref[...]
        ha = jnp.dot(ha_in, W, preferred_element_type=jnp.float32)
        ht = jnp.dot(ht_in, W, preferred_element_type=jnp.float32)
        Ea = jnp.dot(ha, A, preferred_element_type=jnp.float32)
        Et = jnp.dot(ht, A, preferred_element_type=jnp.float32)
        EaT = lax.dot_general(A, ha, (((0,), (1,)), ((), ())),
                              preferred_element_type=jnp.float32)
        EtT = lax.dot_general(A, ht, (((0,), (1,)), ((), ())),
                              preferred_element_type=jnp.float32)
        outs_a = []
        outs_t = []
        for k in range(_H):
            ha_k = ha[:, k * _D:(k + 1) * _D]
            ht_k = ht[:, k * _D:(k + 1) * _D]
            ed_a = Ea[:, 4 + k:5 + k]
            es_t_row = EtT[k:k + 1, :]
            lat = _leaky(ed_a + es_t_row)
            lat = jnp.where(col_mask, lat, _NEG)
            sa = _leaky(Ea[:, k:k + 1] + ed_a)
            m = jnp.maximum(jnp.max(lat, axis=1, keepdims=True), sa)
            ex = jnp.where(col_mask, jnp.exp(lat - m), 0.0)
            exs = jnp.exp(sa - m)
            den = jnp.sum(ex, axis=1, keepdims=True) + exs + 1e-9
            oa = (jnp.dot(ex, ht_k, preferred_element_type=jnp.float32)
                  + exs * ha_k) / den
            outs_a.append(oa)
            ed_t = Et[:, 4 + k:5 + k]
            es_a_row = EaT[k:k + 1, :]
            lta = _leaky(ed_t + es_a_row)
            st = _leaky(Et[:, k:k + 1] + ed_t)
            m2 = jnp.maximum(jnp.max(lta, axis=1, keepdims=True), st)
            ex2 = jnp.exp(lta - m2)
            exs2 = jnp.exp(st - m2)
            den2 = jnp.sum(ex2, axis=1, keepdims=True) + exs2 + 1e-9
            ot = (jnp.dot(ex2, ha_k, preferred_element_type=jnp.float32)
                  + exs2 * ht_k) / den2
            outs_t.append(ot)
        oa = jnp.concatenate(outs_a, axis=1)
        ot = jnp.concatenate(outs_t, axis=1)
        if activate:
            oa = jnp.where(oa > 0, oa, jnp.exp(jnp.minimum(oa, 0.0)) - 1.0)
            ot = jnp.where(ot > 0, ot, jnp.exp(jnp.minimum(ot, 0.0)) - 1.0)
        return oa, ot

    h1a, h1t = layer(a, t, w1_ref, a1_ref, True)
    h2a, h2t = layer(h1a, h1t, w2_ref, a2_ref, False)
    audio_repr = jnp.sum(h2a, axis=0, keepdims=True) / _TA
    text_repr = jnp.sum(jnp.where(row_mask, h2t, 0.0),
                        axis=0, keepdims=True) / _TT
    comb = jnp.concatenate([audio_repr, text_repr], axis=1)
    hmid = jnp.maximum(
        jnp.dot(comb, wm1_ref[...], preferred_element_type=jnp.float32)
        + bm1_ref[...], 0.0)
    out_ref[0] = (jnp.dot(hmid, wm2_ref[...],
                          preferred_element_type=jnp.float32)
                  + bm2_ref[...])


def kernel(audio_feats, text_feats, W1, a_src1, a_dst1, W2, a_src2, a_dst2,
           Wm1, bm1, Wm2, bm2):
    B = audio_feats.shape[0]
    text_p = jnp.pad(text_feats, ((0, 0), (0, _TTP - _TT), (0, 0)))
    sel = jnp.repeat(jnp.eye(_H, dtype=jnp.float32), _D, axis=0)

    def pack(a_s, a_d):
        a_sc = sel * a_s.reshape(-1)[:, None]
        a_dc = sel * a_d.reshape(-1)[:, None]
        pad = jnp.zeros((_H * _D, 128 - 2 * _H), jnp.float32)
        return jnp.concatenate([a_sc, a_dc, pad], axis=1)

    A1 = pack(a_src1, a_dst1)
    A2 = pack(a_src2, a_dst2)

    return pl.pallas_call(
        _gat_fused_kernel,
        grid=(B,),
        in_specs=[
            pl.BlockSpec((1, _TA, _IN), lambda i: (i, 0, 0)),
            pl.BlockSpec((1, _TTP, _IN), lambda i: (i, 0, 0)),
            pl.BlockSpec((_IN, _H * _D), lambda i: (0, 0)),
            pl.BlockSpec((_H * _D, 128), lambda i: (0, 0)),
            pl.BlockSpec((_H * _D, _H * _D), lambda i: (0, 0)),
            pl.BlockSpec((_H * _D, 128), lambda i: (0, 0)),
            pl.BlockSpec((2 * _H * _D, 256), lambda i: (0, 0)),
            pl.BlockSpec((1, 256), lambda i: (0, 0)),
            pl.BlockSpec((256, 256), lambda i: (0, 0)),
            pl.BlockSpec((1, 256), lambda i: (0, 0)),
        ],
        out_specs=pl.BlockSpec((1, 1, 256), lambda i: (i, 0, 0)),
        out_shape=jax.ShapeDtypeStruct((B, 1, 256), jnp.float32),
    )(audio_feats, text_p, W1, A1, W2, A2,
      Wm1, bm1.reshape(1, -1), Wm2, bm2.reshape(1, -1)).reshape(B, 256)


# speedup vs baseline: 131.3998x; 1.0436x over previous
import jax
import jax.numpy as jnp
from jax import lax
from jax.experimental import pallas as pl
from jax.experimental.pallas import tpu as pltpu

_IN = 256
_H = 4
_D = 64
_TA = 200
_TT = 50
_TTP = 64
_NEG = -1e30
_SPG = 2


def _leaky(x):
    return jnp.where(x >= 0, x, 0.2 * x)


def _gat_fused_kernel(a_ref, t_ref, w1_ref, a1_ref, w2_ref, a2_ref,
                      wm1_ref, bm1_ref, wm2_ref, bm2_ref, out_ref):
    col_mask = lax.broadcasted_iota(jnp.int32, (_TA, _TTP), 1) < _TT
    row_mask = lax.broadcasted_iota(jnp.int32, (_TTP, 1), 0) < _TT

    def layer(ha_in, ht_in, w_ref, A_ref, activate):
        W = w_ref[...]
        A = A_ref[...]
        ha = jnp.dot(ha_in, W, preferred_element_type=jnp.float32)
        ht = jnp.dot(ht_in, W, preferred_element_type=jnp.float32)
        Ea = jnp.dot(ha, A, preferred_element_type=jnp.float32)
        Et = jnp.dot(ht, A, preferred_element_type=jnp.float32)
        EaT = lax.dot_general(A, ha, (((0,), (1,)), ((), ())),
                              preferred_element_type=jnp.float32)
        EtT = lax.dot_general(A, ht, (((0,), (1,)), ((), ())),
                              preferred_element_type=jnp.float32)
        outs_a = []
        outs_t = []
        for k in range(_H):
            ha_k = ha[:, k * _D:(k + 1) * _D]
            ht_k = ht[:, k * _D:(k + 1) * _D]
            ed_a = Ea[:, 4 + k:5 + k]
            es_t_row = EtT[k:k + 1, :]
            lat = _leaky(ed_a + es_t_row)
            lat = jnp.where(col_mask, lat, _NEG)
            sa = _leaky(Ea[:, k:k + 1] + ed_a)
            m = jnp.maximum(jnp.max(lat, axis=1, keepdims=True), sa)
            ex = jnp.where(col_mask, jnp.exp(lat - m), 0.0)
            exs = jnp.exp(sa - m)
            den = jnp.sum(ex, axis=1, keepdims=True) + exs + 1e-9
            oa = (jnp.dot(ex, ht_k, preferred_element_type=jnp.float32)
                  + exs * ha_k) / den
            outs_a.append(oa)
            ed_t = Et[:, 4 + k:5 + k]
            es_a_row = EaT[k:k + 1, :]
            lta = _leaky(ed_t + es_a_row)
            st = _leaky(Et[:, k:k + 1] + ed_t)
            m2 = jnp.maximum(jnp.max(lta, axis=1, keepdims=True), st)
            ex2 = jnp.exp(lta - m2)
            exs2 = jnp.exp(st - m2)
            den2 = jnp.sum(ex2, axis=1, keepdims=True) + exs2 + 1e-9
            ot = (jnp.dot(ex2, ha_k, preferred_element_type=jnp.float32)
                  + exs2 * ht_k) / den2
            outs_t.append(ot)
        oa = jnp.concatenate(outs_a, axis=1)
        ot = jnp.concatenate(outs_t, axis=1)
        if activate:
            oa = jnp.where(oa > 0, oa, jnp.exp(jnp.minimum(oa, 0.0)) - 1.0)
            ot = jnp.where(ot > 0, ot, jnp.exp(jnp.minimum(ot, 0.0)) - 1.0)
        return oa, ot

    for s in range(_SPG):
        a = a_ref[s]
        t = t_ref[s]
        h1a, h1t = layer(a, t, w1_ref, a1_ref, True)
        h2a, h2t = layer(h1a, h1t, w2_ref, a2_ref, False)
        audio_repr = jnp.sum(h2a, axis=0, keepdims=True) / _TA
        text_repr = jnp.sum(jnp.where(row_mask, h2t, 0.0),
                            axis=0, keepdims=True) / _TT
        comb = jnp.concatenate([audio_repr, text_repr], axis=1)
        hmid = jnp.maximum(
            jnp.dot(comb, wm1_ref[...], preferred_element_type=jnp.float32)
            + bm1_ref[...], 0.0)
        out_ref[s] = (jnp.dot(hmid, wm2_ref[...],
                              preferred_element_type=jnp.float32)
                      + bm2_ref[...])


def kernel(audio_feats, text_feats, W1, a_src1, a_dst1, W2, a_src2, a_dst2,
           Wm1, bm1, Wm2, bm2):
    B = audio_feats.shape[0]
    text_p = jnp.pad(text_feats, ((0, 0), (0, _TTP - _TT), (0, 0)))
    sel = jnp.repeat(jnp.eye(_H, dtype=jnp.float32), _D, axis=0)

    def pack(a_s, a_d):
        a_sc = sel * a_s.reshape(-1)[:, None]
        a_dc = sel * a_d.reshape(-1)[:, None]
        pad = jnp.zeros((_H * _D, 128 - 2 * _H), jnp.float32)
        return jnp.concatenate([a_sc, a_dc, pad], axis=1)

    A1 = pack(a_src1, a_dst1)
    A2 = pack(a_src2, a_dst2)

    return pl.pallas_call(
        _gat_fused_kernel,
        grid=(B // _SPG,),
        in_specs=[
            pl.BlockSpec((_SPG, _TA, _IN), lambda i: (i, 0, 0)),
            pl.BlockSpec((_SPG, _TTP, _IN), lambda i: (i, 0, 0)),
            pl.BlockSpec((_IN, _H * _D), lambda i: (0, 0)),
            pl.BlockSpec((_H * _D, 128), lambda i: (0, 0)),
            pl.BlockSpec((_H * _D, _H * _D), lambda i: (0, 0)),
            pl.BlockSpec((_H * _D, 128), lambda i: (0, 0)),
            pl.BlockSpec((2 * _H * _D, 256), lambda i: (0, 0)),
            pl.BlockSpec((1, 256), lambda i: (0, 0)),
            pl.BlockSpec((256, 256), lambda i: (0, 0)),
            pl.BlockSpec((1, 256), lambda i: (0, 0)),
        ],
        out_specs=pl.BlockSpec((_SPG, 1, 256), lambda i: (i, 0, 0)),
        out_shape=jax.ShapeDtypeStruct((B, 1, 256), jnp.float32),
    )(audio_feats, text_p, W1, A1, W2, A2,
      Wm1, bm1.reshape(1, -1), Wm2, bm2.reshape(1, -1)).reshape(B, 256)


# speedup vs baseline: 281.4959x; 2.1423x over previous
import jax
import jax.numpy as jnp
from jax import lax
from jax.experimental import pallas as pl
from jax.experimental.pallas import tpu as pltpu

_IN = 256
_H = 4
_D = 64
_TA = 200
_TAP = 256
_TT = 50
_TTP = 64
_NEG = -1e30
_SPG = 2
_F32 = jnp.float32


def _leaky(x):
    return jnp.where(x >= 0, x, 0.2 * x)


def _dotT(a4, x):
    return lax.dot_general(a4, x, (((0,), (1,)), ((), ())),
                           preferred_element_type=_F32)


def _mm(a, b):
    return jnp.dot(a, b, preferred_element_type=_F32)


def _gat_fused_kernel(a_ref, t_ref, w1_ref, p1_ref, w2_ref, p2_ref,
                      re_ref, ret_ref, re2_ref, re2t_ref,
                      wm1_ref, bm1_ref, wm2_ref, bm2_ref, out_ref):
    lane_a = lax.broadcasted_iota(jnp.int32, (_TA, _H * _D), 1)
    mask_a = (lane_a % _D) < _TT
    lane_t = lax.broadcasted_iota(jnp.int32, (_TTP, _H * _TAP), 1)
    mask_t = (lane_t % _TAP) < _TA
    lane_o = lax.broadcasted_iota(jnp.int32, (1, _H * _D), 1) // _D
    row_mask = lax.broadcasted_iota(jnp.int32, (_TTP, 1), 0) < _TT
    RE = re_ref[...]
    RET = ret_ref[...]
    RE2 = re2_ref[...]
    RE2T = re2t_ref[...]
    z56 = jnp.zeros((1, _TAP - _TA), _F32)
    zrow = jnp.zeros((_TAP - _TA, _H * _D), _F32)

    def layer(ha_in, ht_in, w_ref, p_ref, activate):
        W = w_ref[...]
        P = p_ref[...]
        as4 = P[:, 0:4]
        ad4 = P[:, 4:8]
        asd4 = P[:, 8:12]
        ha = _mm(ha_in, W)
        ht = _mm(ht_in, W)

        ed_a4 = _mm(ha, ad4)
        sa_a4 = _leaky(_mm(ha, asd4))
        et_t4 = _dotT(as4, ht)
        es_t_flat = jnp.concatenate(
            [et_t4[k:k + 1, :] for k in range(_H)], axis=1)
        lat = _leaky(_mm(ed_a4, RE) + es_t_flat)
        lat_m = jnp.where(mask_a, lat, _NEG)
        m = jnp.maximum(jnp.max(lat_m, axis=1, keepdims=True),
                        jnp.max(sa_a4, axis=1, keepdims=True))
        ex = jnp.where(mask_a, jnp.exp(lat - m), 0.0)
        exs4 = jnp.exp(sa_a4 - m)
        den4 = _mm(ex, RET) + exs4
        r4 = 1.0 / den4
        htbd = jnp.concatenate(
            [jnp.where(lane_o == k, ht, 0.0) for k in range(_H)], axis=0)
        oa = (_mm(ex, htbd) + _mm(exs4, RE) * ha) * _mm(r4, RE)

        ed_t4 = _mm(ht, ad4)
        sa_t4 = _leaky(_mm(ht, asd4))
        ea_a4 = _dotT(as4, ha)
        es_a_flat = jnp.concatenate(
            sum([[ea_a4[k:k + 1, :], z56] for k in range(_H)], []),
            axis=1)
        lta = _leaky(_mm(ed_t4, RE2) + es_a_flat)
        lta_m = jnp.where(mask_t, lta, _NEG)
        m2 = jnp.maximum(jnp.max(lta_m, axis=1, keepdims=True),
                         jnp.max(sa_t4, axis=1, keepdims=True))
        ex2 = jnp.where(mask_t, jnp.exp(lta - m2), 0.0)
        exs2_4 = jnp.exp(sa_t4 - m2)
        den2_4 = _mm(ex2, RE2T) + exs2_4
        r2_4 = 1.0 / den2_4
        ha_pad = jnp.concatenate([ha, zrow], axis=0)
        habd = jnp.concatenate(
            [jnp.where(lane_o == k, ha_pad, 0.0) for k in range(_H)], axis=0)
        ot = (_mm(ex2, habd) + _mm(exs2_4, RE) * ht) * _mm(r2_4, RE)

        if activate:
            oa = jnp.where(oa > 0, oa, jnp.exp(jnp.minimum(oa, 0.0)) - 1.0)
            ot = jnp.where(ot > 0, ot, jnp.exp(jnp.minimum(ot, 0.0)) - 1.0)
        return oa, ot

    for s in range(_SPG):
        a = a_ref[s]
        t = t_ref[s]
        h1a, h1t = layer(a, t, w1_ref, p1_ref, True)
        h2a, h2t = layer(h1a, h1t, w2_ref, p2_ref, False)
        audio_repr = jnp.sum(h2a, axis=0, keepdims=True) / _TA
        text_repr = jnp.sum(jnp.where(row_mask, h2t, 0.0),
                            axis=0, keepdims=True) / _TT
        comb = jnp.concatenate([audio_repr, text_repr], axis=1)
        hmid = jnp.maximum(_mm(comb, wm1_ref[...]) + bm1_ref[...], 0.0)
        out_ref[s] = _mm(hmid, wm2_ref[...]) + bm2_ref[...]


def kernel(audio_feats, text_feats, W1, a_src1, a_dst1, W2, a_src2, a_dst2,
           Wm1, bm1, Wm2, bm2):
    B = audio_feats.shape[0]
    text_p = jnp.pad(text_feats, ((0, 0), (0, _TTP - _TT), (0, 0)))
    sel = jnp.repeat(jnp.eye(_H, dtype=_F32), _D, axis=0)

    def pack(a_s, a_d):
        a_sc = sel * a_s.reshape(-1)[:, None]
        a_dc = sel * a_d.reshape(-1)[:, None]
        return jnp.concatenate([a_sc, a_dc, a_sc + a_dc], axis=1)

    P1 = pack(a_src1, a_dst1)
    P2 = pack(a_src2, a_dst2)
    eye4 = jnp.eye(_H, dtype=_F32)
    RE = jnp.repeat(eye4, _D, axis=1)
    RE2 = jnp.repeat(eye4, _TAP, axis=1)

    full = lambda shape: pl.BlockSpec(shape, lambda i: tuple(0 for _ in shape))
    return pl.pallas_call(
        _gat_fused_kernel,
        grid=(B // _SPG,),
        in_specs=[
            pl.BlockSpec((_SPG, _TA, _IN), lambda i: (i, 0, 0)),
            pl.BlockSpec((_SPG, _TTP, _IN), lambda i: (i, 0, 0)),
            full((_IN, _H * _D)),
            full((_IN, 12)),
            full((_H * _D, _H * _D)),
            full((_IN, 12)),
            full((_H, _H * _D)),
            full((_H * _D, _H)),
            full((_H, _H * _TAP)),
            full((_H * _TAP, _H)),
            full((2 * _H * _D, 256)),
            full((1, 256)),
            full((256, 256)),
            full((1, 256)),
        ],
        out_specs=pl.BlockSpec((_SPG, 1, 256), lambda i: (i, 0, 0)),
        out_shape=jax.ShapeDtypeStruct((B, 1, 256), _F32),
    )(audio_feats, text_p, W1, P1, W2, P2, RE, RE.T, RE2, RE2.T,
      Wm1, bm1.reshape(1, -1), Wm2, bm2.reshape(1, -1)).reshape(B, 256)


# speedup vs baseline: 282.3829x; 1.0032x over previous
import jax
import jax.numpy as jnp
from jax import lax
from jax.experimental import pallas as pl
from jax.experimental.pallas import tpu as pltpu

_IN = 256
_H = 4
_D = 64
_TA = 200
_TAP = 256
_TT = 50
_TTP = 64
_NEG = -1e30
_SPG = 4
_F32 = jnp.float32


def _leaky(x):
    return jnp.where(x >= 0, x, 0.2 * x)


def _dotT(a4, x):
    return lax.dot_general(a4, x, (((0,), (1,)), ((), ())),
                           preferred_element_type=_F32)


def _mm(a, b):
    return jnp.dot(a, b, preferred_element_type=_F32)


def _gat_fused_kernel(a_ref, t_ref, w1_ref, p1_ref, w2_ref, p2_ref,
                      re_ref, ret_ref, re2_ref, re2t_ref,
                      wm1_ref, bm1_ref, wm2_ref, bm2_ref, out_ref):
    lane_a = lax.broadcasted_iota(jnp.int32, (_TA, _H * _D), 1)
    mask_a = (lane_a % _D) < _TT
    lane_t = lax.broadcasted_iota(jnp.int32, (_TTP, _H * _TAP), 1)
    mask_t = (lane_t % _TAP) < _TA
    lane_o = lax.broadcasted_iota(jnp.int32, (1, _H * _D), 1) // _D
    row_mask = lax.broadcasted_iota(jnp.int32, (_TTP, 1), 0) < _TT
    RE = re_ref[...]
    RET = ret_ref[...]
    RE2 = re2_ref[...]
    RE2T = re2t_ref[...]
    z56 = jnp.zeros((1, _TAP - _TA), _F32)
    zrow = jnp.zeros((_TAP - _TA, _H * _D), _F32)

    def layer(ha_in, ht_in, w_ref, p_ref, activate):
        W = w_ref[...]
        P = p_ref[...]
        as4 = P[:, 0:4]
        ad4 = P[:, 4:8]
        asd4 = P[:, 8:12]
        ha = _mm(ha_in, W)
        ht = _mm(ht_in, W)

        ed_a4 = _mm(ha, ad4)
        sa_a4 = _leaky(_mm(ha, asd4))
        et_t4 = _dotT(as4, ht)
        es_t_flat = jnp.concatenate(
            [et_t4[k:k + 1, :] for k in range(_H)], axis=1)
        lat = _leaky(_mm(ed_a4, RE) + es_t_flat)
        lat_m = jnp.where(mask_a, lat, _NEG)
        m = jnp.maximum(jnp.max(lat_m, axis=1, keepdims=True),
                        jnp.max(sa_a4, axis=1, keepdims=True))
        ex = jnp.where(mask_a, jnp.exp(lat - m), 0.0)
        exs4 = jnp.exp(sa_a4 - m)
        den4 = _mm(ex, RET) + exs4
        r4 = 1.0 / den4
        htbd = jnp.concatenate(
            [jnp.where(lane_o == k, ht, 0.0) for k in range(_H)], axis=0)
        oa = (_mm(ex, htbd) + _mm(exs4, RE) * ha) * _mm(r4, RE)

        ed_t4 = _mm(ht, ad4)
        sa_t4 = _leaky(_mm(ht, asd4))
        ea_a4 = _dotT(as4, ha)
        es_a_flat = jnp.concatenate(
            sum([[ea_a4[k:k + 1, :], z56] for k in range(_H)], []),
            axis=1)
        lta = _leaky(_mm(ed_t4, RE2) + es_a_flat)
        lta_m = jnp.where(mask_t, lta, _NEG)
        m2 = jnp.maximum(jnp.max(lta_m, axis=1, keepdims=True),
                         jnp.max(sa_t4, axis=1, keepdims=True))
        ex2 = jnp.where(mask_t, jnp.exp(lta - m2), 0.0)
        exs2_4 = jnp.exp(sa_t4 - m2)
        den2_4 = _mm(ex2, RE2T) + exs2_4
        r2_4 = 1.0 / den2_4
        ha_pad = jnp.concatenate([ha, zrow], axis=0)
        habd = jnp.concatenate(
            [jnp.where(lane_o == k, ha_pad, 0.0) for k in range(_H)], axis=0)
        ot = (_mm(ex2, habd) + _mm(exs2_4, RE) * ht) * _mm(r2_4, RE)

        if activate:
            oa = jnp.where(oa > 0, oa, jnp.exp(jnp.minimum(oa, 0.0)) - 1.0)
            ot = jnp.where(ot > 0, ot, jnp.exp(jnp.minimum(ot, 0.0)) - 1.0)
        return oa, ot

    for s in range(_SPG):
        a = a_ref[s]
        t = t_ref[s]
        h1a, h1t = layer(a, t, w1_ref, p1_ref, True)
        h2a, h2t = layer(h1a, h1t, w2_ref, p2_ref, False)
        audio_repr = jnp.sum(h2a, axis=0, keepdims=True) / _TA
        text_repr = jnp.sum(jnp.where(row_mask, h2t, 0.0),
                            axis=0, keepdims=True) / _TT
        comb = jnp.concatenate([audio_repr, text_repr], axis=1)
        hmid = jnp.maximum(_mm(comb, wm1_ref[...]) + bm1_ref[...], 0.0)
        out_ref[s] = _mm(hmid, wm2_ref[...]) + bm2_ref[...]


def kernel(audio_feats, text_feats, W1, a_src1, a_dst1, W2, a_src2, a_dst2,
           Wm1, bm1, Wm2, bm2):
    B = audio_feats.shape[0]
    text_p = jnp.pad(text_feats, ((0, 0), (0, _TTP - _TT), (0, 0)))
    sel = jnp.repeat(jnp.eye(_H, dtype=_F32), _D, axis=0)

    def pack(a_s, a_d):
        a_sc = sel * a_s.reshape(-1)[:, None]
        a_dc = sel * a_d.reshape(-1)[:, None]
        return jnp.concatenate([a_sc, a_dc, a_sc + a_dc], axis=1)

    P1 = pack(a_src1, a_dst1)
    P2 = pack(a_src2, a_dst2)
    eye4 = jnp.eye(_H, dtype=_F32)
    RE = jnp.repeat(eye4, _D, axis=1)
    RE2 = jnp.repeat(eye4, _TAP, axis=1)

    full = lambda shape: pl.BlockSpec(shape, lambda i: tuple(0 for _ in shape))
    return pl.pallas_call(
        _gat_fused_kernel,
        grid=(B // _SPG,),
        in_specs=[
            pl.BlockSpec((_SPG, _TA, _IN), lambda i: (i, 0, 0)),
            pl.BlockSpec((_SPG, _TTP, _IN), lambda i: (i, 0, 0)),
            full((_IN, _H * _D)),
            full((_IN, 12)),
            full((_H * _D, _H * _D)),
            full((_IN, 12)),
            full((_H, _H * _D)),
            full((_H * _D, _H)),
            full((_H, _H * _TAP)),
            full((_H * _TAP, _H)),
            full((2 * _H * _D, 256)),
            full((1, 256)),
            full((256, 256)),
            full((1, 256)),
        ],
        out_specs=pl.BlockSpec((_SPG, 1, 256), lambda i: (i, 0, 0)),
        out_shape=jax.ShapeDtypeStruct((B, 1, 256), _F32),
    )(audio_feats, text_p, W1, P1, W2, P2, RE, RE.T, RE2, RE2.T,
      Wm1, bm1.reshape(1, -1), Wm2, bm2.reshape(1, -1)).reshape(B, 256)
